# baseline (speedup 1.0000x reference)
import functools

import jax
import jax.numpy as jnp
from jax import lax
from jax.experimental import pallas as pl
from jax.experimental.pallas import tpu as pltpu
from jax.experimental.pallas import tpu_sc as plsc

B = 16384
K = 32
IMAX = 100000

NUM_WORKERS = 32
BPW = B // NUM_WORKERS
IDX_CHUNK = 128
NCHUNK = BPW // IDX_CHUNK
LANES = 16

PREP_BLK = 512
PREP_GRID = 49
SUPER = PREP_GRID * PREP_BLK
NPACK = 4
TBL_ROWS = NPACK * SUPER


def _prep_body(l0, l1, l2, l3, r0, r1, r2, r3, lout_ref, rt_ref):
    eye = jnp.eye(K, dtype=jnp.float32)
    for cm, ref in enumerate((l0, l1, l2, l3)):
        lout_ref[:, cm * K:(cm + 1) * K] = lax.dot_general(
            ref[...], eye, (((0,), (0,)), ((), ())),
            preferred_element_type=jnp.float32,
            precision=lax.Precision.HIGHEST)
    for cm, ref in enumerate((r0, r1, r2, r3)):
        rt_ref[:, cm * K:(cm + 1) * K] = lax.dot_general(
            ref[...], eye, (((0,), (0,)), ((), ())),
            preferred_element_type=jnp.float32,
            precision=lax.Precision.HIGHEST)


def _prep(Lt, R):
    def strip(cm):
        return pl.BlockSpec((K, PREP_BLK), lambda i, cm=cm: (0, cm * PREP_GRID + i))
    out_spec = pl.BlockSpec((PREP_BLK, NPACK * K), lambda i: (i, 0))
    return pl.pallas_call(
        _prep_body,
        grid=(PREP_GRID,),
        in_specs=[strip(cm) for cm in range(NPACK)] * 2,
        out_specs=[out_spec, out_spec],
        out_shape=[
            jax.ShapeDtypeStruct((SUPER, NPACK * K), jnp.float32),
            jax.ShapeDtypeStruct((SUPER, NPACK * K), jnp.float32),
        ],
    )(Lt, Lt, Lt, Lt, R, R, R, R)


def _sc_body(rows_hbm, cols_hbm, l_hbm, rt_hbm, out_hbm,
             rows_v, cols_v, lg, rg, ob, sem):
    wid = lax.axis_index("s") * 2 + lax.axis_index("c")
    base = wid * BPW

    pltpu.sync_copy(rows_hbm.at[pl.ds(wid * NCHUNK, NCHUNK), :], rows_v)
    pltpu.sync_copy(cols_hbm.at[pl.ds(wid * NCHUNK, NCHUNK), :], cols_v)

    copies = []
    for j in range(NCHUNK):
        copies.append(pltpu.async_copy(
            l_hbm.at[rows_v.at[j]],
            lg.at[pl.ds(j * IDX_CHUNK, IDX_CHUNK), :],
            sem))
        copies.append(pltpu.async_copy(
            rt_hbm.at[cols_v.at[j]],
            rg.at[pl.ds(j * IDX_CHUNK, IDX_CHUNK), :],
            sem))
    for c in copies:
        c.wait()

    def chunk_body(c, carry):
        row_ids = c * LANES + lax.iota(jnp.int32, LANES)
        acc = jnp.zeros((LANES,), jnp.float32)
        for k in range(K):
            kv = jnp.full((LANES,), k, jnp.int32)
            lv = plsc.load_gather(lg, [row_ids, kv])
            rv = plsc.load_gather(rg, [row_ids, kv])
            acc = acc + lv * rv
        ob[pl.ds(c * LANES, LANES)] = acc
        return carry

    lax.fori_loop(0, BPW // LANES, chunk_body, 0)

    pltpu.sync_copy(ob, out_hbm.at[pl.ds(base, BPW)])


def _sc_call(rows, cols, Lsub, Rt):
    mesh = plsc.VectorSubcoreMesh(core_axis_name="c", subcore_axis_name="s")
    f = functools.partial(
        pl.kernel,
        out_type=jax.ShapeDtypeStruct((B,), jnp.float32),
        mesh=mesh,
        scratch_types=[
            pltpu.VMEM((NCHUNK, IDX_CHUNK), jnp.int32),
            pltpu.VMEM((NCHUNK, IDX_CHUNK), jnp.int32),
            pltpu.VMEM((BPW, K), jnp.float32),
            pltpu.VMEM((BPW, K), jnp.float32),
            pltpu.VMEM((BPW,), jnp.float32),
            pltpu.SemaphoreType.DMA,
        ],
        compiler_params=pltpu.CompilerParams(
            needs_layout_passes=False, use_tc_tiling_on_sc=False),
    )(_sc_body)
    return f(rows, cols, Lsub, Rt)


def kernel(indices, L, R):
    rows = indices[0]
    cols = indices[1]
    rows_g = ((rows % SUPER) * NPACK + rows // SUPER).reshape(
        B // IDX_CHUNK, IDX_CHUNK)
    cols_g = ((cols % SUPER) * NPACK + cols // SUPER).reshape(
        B // IDX_CHUNK, IDX_CHUNK)
    Lpack, Rpack = _prep(L.T, R)
    Lsub = Lpack.reshape(TBL_ROWS, K)
    Rt = Rpack.reshape(TBL_ROWS, K)
    return _sc_call(rows_g, cols_g, Lsub, Rt)


# speedup vs baseline: 1.4747x; 1.4747x over previous
import functools

import jax
import jax.numpy as jnp
from jax import lax
from jax.experimental import pallas as pl
from jax.experimental.pallas import tpu as pltpu
from jax.experimental.pallas import tpu_sc as plsc

B = 16384
K = 32
IMAX = 100000

NUM_WORKERS = 32
BPW = B // NUM_WORKERS
IDX_CHUNK = 128
NCHUNK = BPW // IDX_CHUNK
LANES = 16

PREP_BLK = 512
PREP_GRID = 49
SUPER = PREP_GRID * PREP_BLK
NPACK = 4
TBL_ROWS = NPACK * SUPER


def _prep_body(l0, l1, l2, l3, r0, r1, r2, r3, lout_ref, rt_ref):
    for cm, ref in enumerate((l0, l1, l2, l3)):
        lout_ref[:, cm * K:(cm + 1) * K] = ref[...].T
    for cm, ref in enumerate((r0, r1, r2, r3)):
        rt_ref[:, cm * K:(cm + 1) * K] = ref[...].T


def _prep(Lt, R):
    def strip(cm):
        return pl.BlockSpec((K, PREP_BLK), lambda i, cm=cm: (0, cm * PREP_GRID + i))
    out_spec = pl.BlockSpec((PREP_BLK, NPACK * K), lambda i: (i, 0))
    return pl.pallas_call(
        _prep_body,
        grid=(PREP_GRID,),
        in_specs=[strip(cm) for cm in range(NPACK)] * 2,
        out_specs=[out_spec, out_spec],
        out_shape=[
            jax.ShapeDtypeStruct((SUPER, NPACK * K), jnp.float32),
            jax.ShapeDtypeStruct((SUPER, NPACK * K), jnp.float32),
        ],
    )(Lt, Lt, Lt, Lt, R, R, R, R)


def _sc_body(rows_hbm, cols_hbm, l_hbm, rt_hbm, out_hbm,
             rows_v, cols_v, lg, rg, ob, sem):
    wid = lax.axis_index("s") * 2 + lax.axis_index("c")
    base = wid * BPW

    pltpu.sync_copy(rows_hbm.at[pl.ds(wid * NCHUNK, NCHUNK), :], rows_v)
    pltpu.sync_copy(cols_hbm.at[pl.ds(wid * NCHUNK, NCHUNK), :], cols_v)

    copies = []
    for j in range(NCHUNK):
        copies.append(pltpu.async_copy(
            l_hbm.at[rows_v.at[j]],
            lg.at[pl.ds(j * IDX_CHUNK, IDX_CHUNK), :],
            sem))
        copies.append(pltpu.async_copy(
            rt_hbm.at[cols_v.at[j]],
            rg.at[pl.ds(j * IDX_CHUNK, IDX_CHUNK), :],
            sem))
    for c in copies:
        c.wait()

    def chunk_body(c, carry):
        row_ids = c * LANES + lax.iota(jnp.int32, LANES)
        acc = jnp.zeros((LANES,), jnp.float32)
        for k in range(K):
            kv = jnp.full((LANES,), k, jnp.int32)
            lv = plsc.load_gather(lg, [row_ids, kv])
            rv = plsc.load_gather(rg, [row_ids, kv])
            acc = acc + lv * rv
        ob[pl.ds(c * LANES, LANES)] = acc
        return carry

    lax.fori_loop(0, BPW // LANES, chunk_body, 0)

    pltpu.sync_copy(ob, out_hbm.at[pl.ds(base, BPW)])


def _sc_call(rows, cols, Lsub, Rt):
    mesh = plsc.VectorSubcoreMesh(core_axis_name="c", subcore_axis_name="s")
    f = functools.partial(
        pl.kernel,
        out_type=jax.ShapeDtypeStruct((B,), jnp.float32),
        mesh=mesh,
        scratch_types=[
            pltpu.VMEM((NCHUNK, IDX_CHUNK), jnp.int32),
            pltpu.VMEM((NCHUNK, IDX_CHUNK), jnp.int32),
            pltpu.VMEM((BPW, K), jnp.float32),
            pltpu.VMEM((BPW, K), jnp.float32),
            pltpu.VMEM((BPW,), jnp.float32),
            pltpu.SemaphoreType.DMA,
        ],
        compiler_params=pltpu.CompilerParams(
            needs_layout_passes=False, use_tc_tiling_on_sc=False),
    )(_sc_body)
    return f(rows, cols, Lsub, Rt)


def kernel(indices, L, R):
    rows = indices[0]
    cols = indices[1]
    rows_g = ((rows % SUPER) * NPACK + rows // SUPER).reshape(
        B // IDX_CHUNK, IDX_CHUNK)
    cols_g = ((cols % SUPER) * NPACK + cols // SUPER).reshape(
        B // IDX_CHUNK, IDX_CHUNK)
    Lpack, Rpack = _prep(L.T, R)
    Lsub = Lpack.reshape(TBL_ROWS, K)
    Rt = Rpack.reshape(TBL_ROWS, K)
    return _sc_call(rows_g, cols_g, Lsub, Rt)
